# baseline (speedup 1.0000x reference)
import functools

import jax
import jax.numpy as jnp
from jax import lax
from jax.experimental import pallas as pl
from jax.experimental.pallas import tpu as pltpu
from jax.experimental.pallas import tpu_sc as plsc

ENT_ROWS = 1000000
REL_ROWS = 1000
EMB_DIM = 64
BATCH = 16384
NUM_CORES = 2
NUM_SUBCORES = 16
NUM_WORKERS = NUM_CORES * NUM_SUBCORES
BPW = BATCH // NUM_WORKERS
CHUNK = 128
NCHUNKS = BPW // CHUNK

_mesh = plsc.VectorSubcoreMesh(core_axis_name="c", subcore_axis_name="s")


@functools.partial(
    pl.kernel,
    mesh=_mesh,
    out_type=jax.ShapeDtypeStruct((BATCH, EMB_DIM), jnp.float32),
    compiler_params=pltpu.CompilerParams(use_tc_tiling_on_sc=True,
                                         needs_layout_passes=False),
    scratch_types=[
        pltpu.VMEM((BPW,), jnp.int32),
        pltpu.VMEM((BPW,), jnp.int32),
        pltpu.VMEM((BPW,), jnp.int32),
        pltpu.VMEM((CHUNK,), jnp.int32),
        pltpu.VMEM((CHUNK,), jnp.int32),
        pltpu.VMEM((CHUNK,), jnp.int32),
        pltpu.VMEM((CHUNK, 128), jnp.float32),
        pltpu.VMEM((CHUNK, 128), jnp.float32),
        pltpu.VMEM((CHUNK, 128), jnp.float32),
        pltpu.VMEM((CHUNK, EMB_DIM), jnp.float32),
        pltpu.SemaphoreType.DMA,
    ],
)
def _transe_sc(ent_hbm, rel_hbm, head_hbm, ridx_hbm, tail_hbm, out_hbm,
               idxh, idxr, idxt, gih, git, gir, hdst, tdst, rdst, outb, sem):
    wid = lax.axis_index("s") * NUM_CORES + lax.axis_index("c")
    base = wid * BPW

    pltpu.sync_copy(head_hbm.at[pl.ds(base, BPW)], idxh)
    pltpu.sync_copy(ridx_hbm.at[pl.ds(base, BPW)], idxr)
    pltpu.sync_copy(tail_hbm.at[pl.ds(base, BPW)], idxt)

    def do_chunk(chunk, carry):
        cbase = chunk * CHUNK

        halves = []
        for lg in range(CHUNK // 16):
            sl = pl.ds(cbase + lg * 16, 16)
            e_h = idxh[sl]
            e_t = idxt[sl]
            e_r = idxr[sl]
            dsl = pl.ds(lg * 16, 16)
            gih[dsl] = e_h >> 1
            git[dsl] = e_t >> 1
            gir[dsl] = e_r >> 1
            halves.append(((e_h & 1) << 6, (e_t & 1) << 6, (e_r & 1) << 6))

        ch = pltpu.async_copy(ent_hbm.at[gih], hdst, sem)
        ct = pltpu.async_copy(ent_hbm.at[git], tdst, sem)
        cr = pltpu.async_copy(rel_hbm.at[gir], rdst, sem)
        ch.wait()
        ct.wait()
        cr.wait()

        for lg, (o_h, o_t, o_r) in enumerate(halves):
            for j in range(16):
                l = lg * 16 + j
                for k in range(EMB_DIM // 16):
                    hv = hdst[l, pl.ds(o_h[j] + k * 16, 16)]
                    tv = tdst[l, pl.ds(o_t[j] + k * 16, 16)]
                    rv = rdst[l, pl.ds(o_r[j] + k * 16, 16)]
                    outb[l, pl.ds(k * 16, 16)] = hv + rv - tv

        pltpu.sync_copy(outb, out_hbm.at[pl.ds(base + cbase, CHUNK)])
        return carry

    lax.fori_loop(0, NCHUNKS, do_chunk, 0)


def kernel(head, relation, tail, ent_emb, rel_emb):
    return _transe_sc(
        ent_emb.reshape(ENT_ROWS // 2, 128),
        rel_emb.reshape(REL_ROWS // 2, 128),
        head.reshape(BATCH),
        relation.reshape(BATCH),
        tail.reshape(BATCH),
    )


# speedup vs baseline: 1.9548x; 1.9548x over previous
import functools

import jax
import jax.numpy as jnp
from jax import lax
from jax.experimental import pallas as pl
from jax.experimental.pallas import tpu as pltpu
from jax.experimental.pallas import tpu_sc as plsc

ENT_ROWS = 1000000
REL_ROWS = 1000
EMB_DIM = 64
BATCH = 16384
NUM_CORES = 2
NUM_SUBCORES = 16
NUM_WORKERS = NUM_CORES * NUM_SUBCORES
BPW = BATCH // NUM_WORKERS
CHUNK = 32
NCHUNKS = BPW // CHUNK

_mesh = plsc.VectorSubcoreMesh(core_axis_name="c", subcore_axis_name="s")


@functools.partial(
    pl.kernel,
    mesh=_mesh,
    out_type=jax.ShapeDtypeStruct((BATCH, EMB_DIM), jnp.float32),
    compiler_params=pltpu.CompilerParams(use_tc_tiling_on_sc=True,
                                         needs_layout_passes=False),
    scratch_types=[
        pltpu.VMEM((BPW,), jnp.int32),
        pltpu.VMEM((BPW,), jnp.int32),
        pltpu.VMEM((BPW,), jnp.int32),
        pltpu.VMEM((CHUNK,), jnp.int32),
        pltpu.VMEM((CHUNK,), jnp.int32),
        pltpu.VMEM((CHUNK,), jnp.int32),
        pltpu.VMEM((CHUNK * 8, EMB_DIM), jnp.float32),
        pltpu.VMEM((CHUNK * 8, EMB_DIM), jnp.float32),
        pltpu.VMEM((CHUNK * 8, EMB_DIM), jnp.float32),
        pltpu.VMEM((CHUNK, EMB_DIM), jnp.float32),
        pltpu.SemaphoreType.DMA,
    ],
)
def _transe_sc(ent_hbm, rel_hbm, head_hbm, ridx_hbm, tail_hbm, out_hbm,
               idxh, idxr, idxt, gih, git, gir, hdst, tdst, rdst, outb, sem):
    wid = lax.axis_index("s") * NUM_CORES + lax.axis_index("c")
    base = wid * BPW

    pltpu.sync_copy(head_hbm.at[pl.ds(base, BPW)], idxh)
    pltpu.sync_copy(ridx_hbm.at[pl.ds(base, BPW)], idxr)
    pltpu.sync_copy(tail_hbm.at[pl.ds(base, BPW)], idxt)

    iota = lax.iota(jnp.int32, 16)

    def do_chunk(chunk, carry):
        cbase = chunk * CHUNK

        subs = []
        copies = []
        for lg in range(CHUNK // 16):
            sl = pl.ds(cbase + lg * 16, 16)
            e_h = idxh[sl]
            e_t = idxt[sl]
            e_r = idxr[sl]
            th = e_h >> 3
            tt = e_t >> 3
            tr = e_r >> 3
            for j in range(16):
                dsl = pl.ds((lg * 16 + j) * 8, 8)
                copies.append(pltpu.async_copy(
                    ent_hbm.at[th[j], :, :], hdst.at[dsl, :], sem))
                copies.append(pltpu.async_copy(
                    ent_hbm.at[tt[j], :, :], tdst.at[dsl, :], sem))
                copies.append(pltpu.async_copy(
                    rel_hbm.at[tr[j], :, :], rdst.at[dsl, :], sem))
            subs.append((e_h & 7, e_t & 7, e_r & 7))
        for c in copies:
            c.wait()

        def extract(d, c):
            cols = d * 16 + iota
            for lg, (s_h, s_t, s_r) in enumerate(subs):
                for j in range(16):
                    l = lg * 16 + j
                    hv = plsc.load_gather(
                        hdst, [jnp.broadcast_to(l * 8 + s_h[j], (16,)), cols])
                    tv = plsc.load_gather(
                        tdst, [jnp.broadcast_to(l * 8 + s_t[j], (16,)), cols])
                    rv = plsc.load_gather(
                        rdst, [jnp.broadcast_to(l * 8 + s_r[j], (16,)), cols])
                    outb[l, pl.ds(d * 16, 16)] = hv + rv - tv
            return c

        lax.fori_loop(0, EMB_DIM // 16, extract, 0)

        pltpu.sync_copy(outb, out_hbm.at[pl.ds(base + cbase, CHUNK)])
        return carry

    lax.fori_loop(0, NCHUNKS, do_chunk, 0)


def kernel(head, relation, tail, ent_emb, rel_emb):
    return _transe_sc(
        ent_emb.reshape(ENT_ROWS // 8, 8, EMB_DIM),
        rel_emb.reshape(REL_ROWS // 8, 8, EMB_DIM),
        head.reshape(BATCH),
        relation.reshape(BATCH),
        tail.reshape(BATCH),
    )


# speedup vs baseline: 2.1796x; 1.1150x over previous
import functools

import jax
import jax.numpy as jnp
from jax import lax
from jax.experimental import pallas as pl
from jax.experimental.pallas import tpu as pltpu
from jax.experimental.pallas import tpu_sc as plsc

ENT_ROWS = 1000000
REL_ROWS = 1000
EMB_DIM = 64
BATCH = 16384
NUM_CORES = 2
NUM_SUBCORES = 16
NUM_WORKERS = NUM_CORES * NUM_SUBCORES
BPW = BATCH // NUM_WORKERS
CHUNK = 16
NCHUNKS = BPW // CHUNK
NPAIRS = NCHUNKS // 2

_mesh = plsc.VectorSubcoreMesh(core_axis_name="c", subcore_axis_name="s")


@functools.partial(
    pl.kernel,
    mesh=_mesh,
    out_type=jax.ShapeDtypeStruct((BATCH, EMB_DIM), jnp.float32),
    compiler_params=pltpu.CompilerParams(use_tc_tiling_on_sc=True,
                                         needs_layout_passes=False),
    scratch_types=[
        pltpu.VMEM((BPW,), jnp.int32),
        pltpu.VMEM((BPW,), jnp.int32),
        pltpu.VMEM((BPW,), jnp.int32),
        pltpu.VMEM((CHUNK,), jnp.int32),
        pltpu.VMEM((CHUNK,), jnp.int32),
        pltpu.VMEM((CHUNK * 8, EMB_DIM), jnp.float32),
        pltpu.VMEM((CHUNK * 8, EMB_DIM), jnp.float32),
        pltpu.VMEM((CHUNK, 128), jnp.float32),
        pltpu.VMEM((CHUNK * 8, EMB_DIM), jnp.float32),
        pltpu.VMEM((CHUNK * 8, EMB_DIM), jnp.float32),
        pltpu.VMEM((CHUNK, 128), jnp.float32),
        pltpu.VMEM((CHUNK, EMB_DIM), jnp.float32),
        pltpu.SemaphoreType.DMA,
        pltpu.SemaphoreType.DMA,
    ],
)
def _transe_sc(ent_hbm, rel_hbm, head_hbm, ridx_hbm, tail_hbm, out_hbm,
               idxh, idxr, idxt, girA, girB, hA, tA, rA, hB, tB, rB,
               outb, semA, semB):
    wid = lax.axis_index("s") * NUM_CORES + lax.axis_index("c")
    base = wid * BPW

    pltpu.sync_copy(head_hbm.at[pl.ds(base, BPW)], idxh)
    pltpu.sync_copy(ridx_hbm.at[pl.ds(base, BPW)], idxr)
    pltpu.sync_copy(tail_hbm.at[pl.ds(base, BPW)], idxt)

    iota = lax.iota(jnp.int32, 16)

    def enqueue(cidx, hb, tb, rb, gir, sem):
        sl = pl.ds(cidx * CHUNK, CHUNK)
        th = idxh[sl] >> 3
        tt = idxt[sl] >> 3
        gir[...] = idxr[sl] >> 1
        pltpu.async_copy(rel_hbm.at[gir], rb, sem)
        for j in range(CHUNK):
            dsl = pl.ds(j * 8, 8)
            pltpu.async_copy(ent_hbm.at[th[j], :, :], hb.at[dsl, :], sem)
            pltpu.async_copy(ent_hbm.at[tt[j], :, :], tb.at[dsl, :], sem)

    def drain(hb, tb, rb, sem):
        dummy = out_hbm.at[pl.ds(0, CHUNK * 8), :]
        pltpu.make_async_copy(dummy, hb, sem).wait()
        pltpu.make_async_copy(dummy, tb, sem).wait()
        pltpu.make_async_copy(rel_hbm.at[pl.ds(0, CHUNK)], rb, sem).wait()

    def extract_write(cidx, hb, tb, rb):
        sl = pl.ds(cidx * CHUNK, CHUNK)
        s_h = idxh[sl] & 7
        s_t = idxt[sl] & 7
        o_r = (idxr[sl] & 1) << 6

        def extract(d, c):
            cols = d * 16 + iota
            for j in range(CHUNK):
                hv = plsc.load_gather(
                    hb, [jnp.broadcast_to(j * 8 + s_h[j], (16,)), cols])
                tv = plsc.load_gather(
                    tb, [jnp.broadcast_to(j * 8 + s_t[j], (16,)), cols])
                rv = rb[j, pl.ds(o_r[j] + d * 16, 16)]
                outb[j, pl.ds(d * 16, 16)] = hv + rv - tv
            return c

        lax.fori_loop(0, EMB_DIM // 16, extract, 0)
        pltpu.sync_copy(outb, out_hbm.at[pl.ds(base + cidx * CHUNK, CHUNK)])

    enqueue(0, hA, tA, rA, girA, semA)

    def pairbody(p, carry):
        c0 = p * 2
        enqueue(c0 + 1, hB, tB, rB, girB, semB)
        drain(hA, tA, rA, semA)
        extract_write(c0, hA, tA, rA)

        @pl.when(p < NPAIRS - 1)
        def _():
            enqueue(c0 + 2, hA, tA, rA, girA, semA)

        drain(hB, tB, rB, semB)
        extract_write(c0 + 1, hB, tB, rB)
        return carry

    lax.fori_loop(0, NPAIRS, pairbody, 0)


def kernel(head, relation, tail, ent_emb, rel_emb):
    return _transe_sc(
        ent_emb.reshape(ENT_ROWS // 8, 8, EMB_DIM),
        rel_emb.reshape(REL_ROWS // 2, 128),
        head.reshape(BATCH),
        relation.reshape(BATCH),
        tail.reshape(BATCH),
    )
